# baseline (speedup 1.0000x reference)
import functools

import jax
import jax.numpy as jnp
from jax import lax
from jax.experimental import pallas as pl
from jax.experimental.pallas import tpu as pltpu
from jax.experimental.pallas import tpu_sc as plsc

NC = 1000
B0 = 4096
B1 = 26
CLSB = 25
NK = NC // CLSB
BW = 2048
NH = B0 // BW
PER_SLAB = NK * NH
TOT = B1 * PER_SLAB
NW = 32
NT = TOT // NW
NGRP = BW // 16
BRICK = CLSB * BW


def _worker(x_hbm, out_hbm, buf0, buf1, xc0, xc1, sem0, sem1):
    wid = lax.axis_index("s") * 2 + lax.axis_index("c")
    iota16 = lax.iota(jnp.int32, 16)
    ones = jnp.full((16,), 1.0, jnp.float32)
    zeros16 = jnp.zeros((16,), jnp.float32)

    def _zero(buf):
        def zb(i, _):
            for u in range(8):
                buf[pl.ds(i * 128 + u * 16, 16)] = zeros16
            return 0

        lax.fori_loop(0, BRICK // 128, zb, 0)

    _zero(buf0)
    _zero(buf1)

    def _params(bid):
        j = bid // PER_SLAB
        rem = bid - j * PER_SLAB
        half = rem // NK
        k = rem - half * NK
        cls0 = k * CLSB
        row0 = j * NC + cls0
        col0 = half * BW
        return j, cls0, row0, col0

    def _scan(buf, xc, cls0, val):
        def gb(gi, _):
            for u in range(8):
                off = gi * 128 + u * 16
                xv = xc[pl.ds(off, 16)]
                rows = xv - cls0
                m = (rows >= 0) & (rows < CLSB)
                flat = rows * BW + (iota16 + off)
                plsc.store_scatter(buf, [flat], val, mask=m)
            return 0

        lax.fori_loop(0, NGRP // 8, gb, 0)

    def _rows(buf, sem, row0, col0, fire):
        def rb(r, _):
            copy = pltpu.make_async_copy(
                buf.at[pl.ds(r * BW, BW)],
                out_hbm.at[row0 + r, pl.ds(col0, BW)],
                sem,
            )
            if fire:
                copy.start()
            else:
                copy.wait()
            return 0

        lax.fori_loop(0, CLSB, rb, 0)

    def _phase(t, buf, xc, sem):
        @pl.when(t < NT)
        def _():
            bid = wid * NT + t
            j, cls0, row0, col0 = _params(bid)

            @pl.when(t >= 2)
            def _unset():
                jo, cls0o, row0o, col0o = _params(bid - 2)
                _rows(buf, sem, row0o, col0o, fire=False)
                _scan(buf, xc, cls0o, zeros16)

            pltpu.sync_copy(x_hbm.at[pl.ds(j * B0 + col0, BW)], xc)
            _scan(buf, xc, cls0, ones)
            _rows(buf, sem, row0, col0, fire=True)

    def tb(i, _):
        _phase(2 * i, buf0, xc0, sem0)
        _phase(2 * i + 1, buf1, xc1, sem1)
        return 0

    lax.fori_loop(0, (NT + 2) // 2, tb, 0)

    jd, cd, rd, cold = _params(wid * NT + NT - 1)
    _rows(buf0, sem0, rd, cold, fire=False)
    jd, cd, rd, cold = _params(wid * NT + NT - 2)
    _rows(buf1, sem1, rd, cold, fire=False)


_mesh = plsc.VectorSubcoreMesh(core_axis_name="c", subcore_axis_name="s")


@jax.jit
def kernel(x):
    xt = x.astype(jnp.int32).T.reshape(B1 * B0)
    sc = functools.partial(
        pl.kernel,
        out_type=jax.ShapeDtypeStruct((B1 * NC, B0), jnp.float32),
        mesh=_mesh,
        compiler_params=pltpu.CompilerParams(needs_layout_passes=False),
        scratch_types=[
            pltpu.VMEM((BRICK,), jnp.float32),
            pltpu.VMEM((BRICK,), jnp.float32),
            pltpu.VMEM((BW,), jnp.int32),
            pltpu.VMEM((BW,), jnp.int32),
            pltpu.SemaphoreType.DMA,
            pltpu.SemaphoreType.DMA,
        ],
    )(_worker)
    out2d = sc(xt)
    return jnp.transpose(out2d.reshape(B1, NC, B0), (2, 0, 1))


# speedup vs baseline: 1.9127x; 1.9127x over previous
import jax
import jax.numpy as jnp
from jax.experimental import pallas as pl
from jax.experimental.pallas import tpu as pltpu

NC = 1000
B0 = 4096
B1 = 26
NBUF = 3


def _onehot_body(x_ref, out_ref, scratch, sems):
    i = pl.program_id(0)
    b = jax.lax.rem(i, NBUF)

    def mkcopy(bb, j):
        return pltpu.make_async_copy(
            scratch.at[bb], out_ref.at[pl.ds(j, 1)], sems.at[bb]
        )

    @pl.when(i >= NBUF)
    def _wait_prev():
        mkcopy(b, i).wait()

    iota = jax.lax.broadcasted_iota(jnp.int32, (1, NC, B0), 1)
    xv = x_ref[:, :, :]
    scratch[b] = (iota == xv).astype(jnp.float32)
    mkcopy(b, i).start()

    @pl.when(i == B1 - 1)
    def _drain():
        for bb in range(NBUF):
            mkcopy(bb, i).wait()


def kernel(x):
    xt = x.astype(jnp.int32).T.reshape(B1, 1, B0)
    out_t = pl.pallas_call(
        _onehot_body,
        grid=(B1,),
        in_specs=[pl.BlockSpec((1, 1, B0), lambda j: (j, 0, 0))],
        out_specs=pl.BlockSpec(memory_space=pl.ANY),
        out_shape=jax.ShapeDtypeStruct((B1, NC, B0), jnp.float32),
        scratch_shapes=[
            pltpu.VMEM((NBUF, 1, NC, B0), jnp.float32),
            pltpu.SemaphoreType.DMA((NBUF,)),
        ],
    )(xt)
    return jnp.transpose(out_t, (2, 0, 1))


# speedup vs baseline: 1.9449x; 1.0168x over previous
import jax
import jax.numpy as jnp
from jax.experimental import pallas as pl
from jax.experimental.pallas import tpu as pltpu

NC = 1000
B0 = 4096
B1 = 26
NBUF = 3


def _onehot_body(x_ref, out_ref, scratch, sems):
    i = pl.program_id(0)
    b = jax.lax.rem(i, NBUF)

    def mkcopy(bb, j):
        return pltpu.make_async_copy(
            scratch.at[bb], out_ref.at[pl.ds(j, 1)], sems.at[bb]
        )

    @pl.when(i >= NBUF)
    def _wait_prev():
        mkcopy(b, i).wait()

    iota = jax.lax.broadcasted_iota(jnp.int32, (1, NC, B0), 1)
    xv = x_ref[:, pl.ds(i, 1), :]
    scratch[b] = (iota == xv).astype(jnp.float32)
    mkcopy(b, i).start()

    @pl.when(i == B1 - 1)
    def _drain():
        for bb in range(NBUF):
            mkcopy(bb, i).wait()


def kernel(x):
    xt = x.astype(jnp.int32).T.reshape(1, B1, B0)
    out_t = pl.pallas_call(
        _onehot_body,
        grid=(B1,),
        in_specs=[pl.BlockSpec((1, B1, B0), lambda j: (0, 0, 0))],
        out_specs=pl.BlockSpec(memory_space=pl.ANY),
        out_shape=jax.ShapeDtypeStruct((B1, NC, B0), jnp.float32),
        scratch_shapes=[
            pltpu.VMEM((NBUF, 1, NC, B0), jnp.float32),
            pltpu.SemaphoreType.DMA((NBUF,)),
        ],
    )(xt)
    return jnp.transpose(out_t, (2, 0, 1))


# speedup vs baseline: 1.9758x; 1.0159x over previous
import jax
import jax.numpy as jnp
from jax.experimental import pallas as pl
from jax.experimental.pallas import tpu as pltpu

NC = 1000
B0 = 4096
B1 = 26
LW = 2048
NLB = B0 // LW
NSTEP = B1 * NLB
NBUF = 4


def _onehot_body(x_ref, out_ref, scratch, sems):
    i = pl.program_id(0)
    j = i // NLB
    h = jax.lax.rem(i, NLB)
    b = jax.lax.rem(i, NBUF)

    def mkcopy(bb, jj, hh):
        return pltpu.make_async_copy(
            scratch.at[bb],
            out_ref.at[pl.ds(jj, 1), :, pl.ds(hh * LW, LW)],
            sems.at[bb],
        )

    @pl.when(i >= NBUF)
    def _wait_prev():
        mkcopy(b, j, h).wait()

    iota = jax.lax.broadcasted_iota(jnp.int32, (1, NC, LW), 1)
    xv = x_ref[:, pl.ds(j, 1), pl.ds(h * LW, LW)]
    scratch[b] = (iota == xv).astype(jnp.float32)
    mkcopy(b, j, h).start()

    @pl.when(i == NSTEP - 1)
    def _drain():
        for bb in range(NBUF):
            mkcopy(bb, j, h).wait()


def kernel(x):
    xt = x.astype(jnp.int32).T.reshape(1, B1, B0)
    out_t = pl.pallas_call(
        _onehot_body,
        grid=(NSTEP,),
        in_specs=[pl.BlockSpec((1, B1, B0), lambda j: (0, 0, 0))],
        out_specs=pl.BlockSpec(memory_space=pl.ANY),
        out_shape=jax.ShapeDtypeStruct((B1, NC, B0), jnp.float32),
        scratch_shapes=[
            pltpu.VMEM((NBUF, 1, NC, LW), jnp.float32),
            pltpu.SemaphoreType.DMA((NBUF,)),
        ],
    )(xt)
    return jnp.transpose(out_t, (2, 0, 1))
